# baseline (speedup 1.0000x reference)
import functools

import jax
import jax.numpy as jnp
from jax import lax
from jax.experimental import pallas as pl
from jax.experimental.pallas import tpu as pltpu
from jax.experimental.pallas import tpu_sc as plsc

NC = 2
NS = 16
NW = NC * NS
K = 128


def _cdiv(a, b):
    return (a + b - 1) // b


def _round_up(a, b):
    return _cdiv(a, b) * b


def _make_deg_kernel(Npad, n_chunks, rpt):
    per_w = _cdiv(n_chunks, NW)
    mesh = plsc.VectorSubcoreMesh(core_axis_name="c", subcore_axis_name="s")

    @functools.partial(
        pl.kernel,
        out_type=jax.ShapeDtypeStruct((NC * Npad,), jnp.float32),
        mesh=mesh,
        scratch_types=[
            pltpu.VMEM((K,), jnp.int32),
            pltpu.VMEM((K,), jnp.float32),
            pltpu.VMEM((rpt,), jnp.float32),
            pltpu.VMEM_SHARED((Npad,), jnp.float32),
        ],
    )
    def deg_kernel(dst_hbm, ones_hbm, zeros_hbm, out_hbm, dst_v, ones_v, row_v, acc_sh):
        c = lax.axis_index("c")
        s = lax.axis_index("s")
        wid = s * NC + c
        row0 = s * rpt
        pltpu.sync_copy(zeros_hbm, row_v)
        pltpu.sync_copy(row_v, acc_sh.at[pl.ds(row0, rpt)])
        pltpu.sync_copy(ones_hbm, ones_v)
        plsc.subcore_barrier()

        def body(j, carry):
            chunk = j * NW + wid

            @pl.when(chunk < n_chunks)
            def _():
                pltpu.sync_copy(dst_hbm.at[chunk], dst_v)
                pltpu.sync_copy(ones_v, acc_sh.at[dst_v], add=True)

            return carry

        lax.fori_loop(0, per_w, body, None)
        plsc.subcore_barrier()
        pltpu.sync_copy(acc_sh.at[pl.ds(row0, rpt)], row_v)
        pltpu.sync_copy(row_v, out_hbm.at[pl.ds(c * Npad + row0, rpt)])

    return deg_kernel


def _make_agg_kernel(Npad, D, n_chunks, rpt, nb=2):
    per_w = _round_up(_cdiv(n_chunks, NW), max(8, nb))
    if per_w <= 64:
        slabs = [(0, per_w)]
        slab_max = per_w
    else:
        half = _round_up(per_w // 2, nb)
        slabs = [(0, half), (half, per_w - half)]
        slab_max = half
    mesh = plsc.VectorSubcoreMesh(core_axis_name="c", subcore_axis_name="s")

    scratch = (
        [pltpu.VMEM((slab_max, K), jnp.int32),
         pltpu.VMEM((slab_max, K), jnp.int32)]
        + [pltpu.VMEM((K, D), jnp.float32) for _ in range(nb)]
        + [pltpu.VMEM_SHARED((Npad, D), jnp.float32)]
        + [pltpu.SemaphoreType.DMA for _ in range(2 * nb)]
    )

    @functools.partial(
        pl.kernel,
        out_type=jax.ShapeDtypeStruct((NC, Npad, D), jnp.float32),
        mesh=mesh,
        scratch_types=scratch,
    )
    def agg_kernel(h_hbm, src_hbm, dst_hbm, zeros_hbm, out_hbm, src_vb, dst_vb,
                   *rest):
        rows = rest[:nb]
        acc_sh = rest[nb]
        sem_g = rest[nb + 1:nb + 1 + nb]
        sem_s = rest[nb + 1 + nb:]
        c = lax.axis_index("c")
        s = lax.axis_index("s")
        wid = s * NC + c
        row0 = s * rpt
        chunk0 = wid * per_w
        chunk_end = jnp.minimum(chunk0 + per_w, n_chunks)

        pltpu.sync_copy(zeros_hbm, acc_sh.at[pl.ds(row0, rpt)])
        plsc.subcore_barrier()

        def gather(b, ql):
            return pltpu.make_async_copy(h_hbm.at[src_vb.at[ql]], rows[b],
                                         sem_g[b])

        def scatter(b, ql):
            return pltpu.make_async_copy(rows[b], acc_sh.at[dst_vb.at[ql]],
                                         sem_s[b])

        for seg0, seg_len in slabs:
            pltpu.sync_copy(src_hbm.at[pl.ds(chunk0 + seg0, seg_len)],
                            src_vb.at[pl.ds(0, seg_len)])
            pltpu.sync_copy(dst_hbm.at[pl.ds(chunk0 + seg0, seg_len)],
                            dst_vb.at[pl.ds(0, seg_len)])

            for b in range(nb):
                @pl.when(chunk0 + seg0 + b < chunk_end)
                def _(b=b):
                    gather(b, b).start()

            def body(r, carry, seg0=seg0, seg_len=seg_len):
                for b in range(nb):
                    ql = r * nb + b
                    q = seg0 + ql

                    @pl.when(chunk0 + q < chunk_end)
                    def _(b=b, ql=ql):
                        gather(b, ql).wait()
                        scatter(b, ql).start(add=True)

                for b in range(nb):
                    ql = r * nb + b
                    q = seg0 + ql
                    qln = ql + nb

                    @pl.when(chunk0 + q < chunk_end)
                    def _(b=b, ql=ql):
                        scatter(b, ql).wait()

                    @pl.when((qln < seg_len) & (chunk0 + seg0 + qln < chunk_end))
                    def _(b=b, qln=qln):
                        gather(b, qln).start()

                return carry

            lax.fori_loop(0, seg_len // nb, body, None)

        plsc.subcore_barrier()
        pltpu.sync_copy(acc_sh.at[pl.ds(row0, rpt)], out_hbm.at[c, pl.ds(row0, rpt)])

    return agg_kernel


def _tc_matmul(xp, W1):
    Npad, DIN = xp.shape
    DH = W1.shape[1]

    def body(x_ref, w_ref, y_ref):
        y_ref[...] = jnp.dot(x_ref[...], w_ref[...],
                             preferred_element_type=jnp.float32)

    return pl.pallas_call(
        body,
        out_shape=jax.ShapeDtypeStruct((Npad, DH), jnp.float32),
    )(xp, W1)


def _tc_dinv_scale(degp0, degp1, y1):
    Npad, DH = y1.shape

    def body(d0_ref, d1_ref, y_ref, dinv_ref, ys_ref):
        deg = d0_ref[...] + d1_ref[...] + 1.0
        dinv = lax.rsqrt(deg)
        dinv_ref[...] = dinv
        ys_ref[...] = y_ref[...] * dinv

    return pl.pallas_call(
        body,
        out_shape=[
            jax.ShapeDtypeStruct((Npad, 1), jnp.float32),
            jax.ShapeDtypeStruct((Npad, DH), jnp.float32),
        ],
    )(degp0, degp1, y1)


def _tc_relu(p0, p1, y1, dinv, b1):
    Npad, DH = y1.shape

    def body(p0_ref, p1_ref, y_ref, dinv_ref, b_ref, out_ref):
        dinv = dinv_ref[...]
        t = p0_ref[...] + p1_ref[...] + y_ref[...] * dinv
        z = jnp.maximum(dinv * t + b_ref[...], 0.0)
        out_ref[...] = z * dinv

    return pl.pallas_call(
        body,
        out_shape=jax.ShapeDtypeStruct((Npad, DH), jnp.float32),
    )(p0, p1, y1, dinv, b1)


def _tc_logsoftmax(p0, p1, zs, dinv, W2p, b2, dout):
    Npad, DH = zs.shape
    Dp2 = W2p.shape[1]

    def body(p0_ref, p1_ref, z_ref, dinv_ref, w_ref, b_ref, out_ref):
        agg = dinv_ref[...] * (p0_ref[...] + p1_ref[...] + z_ref[...])
        t = jnp.dot(agg, w_ref[...], preferred_element_type=jnp.float32)
        t = t + b_ref[...]
        col = lax.broadcasted_iota(jnp.int32, t.shape, 1)
        valid = col < dout
        t = jnp.where(valid, t, jnp.float32(-1e30))
        m = jnp.max(t, axis=1, keepdims=True)
        e = jnp.where(valid, jnp.exp(t - m), 0.0)
        lse = jnp.log(jnp.sum(e, axis=1, keepdims=True))
        out_ref[...] = t - m - lse

    return pl.pallas_call(
        body,
        out_shape=jax.ShapeDtypeStruct((Npad, Dp2), jnp.float32),
    )(p0, p1, zs, dinv, W2p, b2)


def kernel(x, edge_index, W1, b1, W2, b2):
    N, DIN = x.shape
    DH = W1.shape[1]
    DOUT = W2.shape[1]
    E = edge_index.shape[1]

    Npad = _cdiv(N, 128) * 128
    rpt = Npad // NS
    n_chunks = _cdiv(E, K)
    Ep = n_chunks * K
    Dp2 = _cdiv(DOUT, 16) * 16
    nb = 2

    src = edge_index[0].astype(jnp.int32)
    dst = edge_index[1].astype(jnp.int32)
    if Ep != E:
        pad = N + (jnp.arange(Ep - E, dtype=jnp.int32) % (Npad - N))
        src = jnp.concatenate([src, pad])
        dst = jnp.concatenate([dst, pad])
    src2 = src.reshape(n_chunks, K)
    dst2 = dst.reshape(n_chunks, K)
    per_w = _round_up(_cdiv(n_chunks, NW), max(8, nb))
    n_chunks_pad = NW * per_w
    if n_chunks_pad != n_chunks:
        src2 = jnp.pad(src2, ((0, n_chunks_pad - n_chunks), (0, 0)))
        dst2 = jnp.pad(dst2, ((0, n_chunks_pad - n_chunks), (0, 0)))

    xp = jnp.pad(x, ((0, Npad - N), (0, 0)))
    W2p = jnp.pad(W2, ((0, 0), (0, Dp2 - DOUT)))
    b1r = b1.reshape(1, DH)
    b2r = jnp.pad(b2, (0, Dp2 - DOUT)).reshape(1, Dp2)
    ones_k = jnp.ones((K,), jnp.float32)
    zeros_deg = jnp.zeros((rpt,), jnp.float32)
    zeros_h = jnp.zeros((rpt, DH), jnp.float32)

    y1 = _tc_matmul(xp, W1)
    degp = _make_deg_kernel(Npad, n_chunks, rpt)(dst2, ones_k, zeros_deg)
    degp0 = degp[:Npad].reshape(Npad, 1)
    degp1 = degp[Npad:].reshape(Npad, 1)

    dinv, ys = _tc_dinv_scale(degp0, degp1, y1)

    agg_fn = _make_agg_kernel(Npad, DH, n_chunks, rpt, nb=nb)
    aggp = agg_fn(ys, src2, dst2, zeros_h)
    zs = _tc_relu(aggp[0], aggp[1], y1, dinv, b1r)

    agg2 = agg_fn(zs, src2, dst2, zeros_h)
    out = _tc_logsoftmax(agg2[0], agg2[1], zs, dinv, W2p, b2r, DOUT)

    return out[:N, :DOUT]


# speedup vs baseline: 1.0998x; 1.0998x over previous
import functools

import jax
import jax.numpy as jnp
from jax import lax
from jax.experimental import pallas as pl
from jax.experimental.pallas import tpu as pltpu
from jax.experimental.pallas import tpu_sc as plsc

NC = 2
NS = 16
NW = NC * NS
K = 128


def _cdiv(a, b):
    return (a + b - 1) // b


def _round_up(a, b):
    return _cdiv(a, b) * b


def _make_deg_kernel(Npad, n_chunks, rpt, nb=4):
    per_w = _round_up(_cdiv(n_chunks, NW), nb)
    mesh = plsc.VectorSubcoreMesh(core_axis_name="c", subcore_axis_name="s")

    @functools.partial(
        pl.kernel,
        out_type=jax.ShapeDtypeStruct((NC * Npad,), jnp.float32),
        mesh=mesh,
        scratch_types=[
            pltpu.VMEM((per_w, K), jnp.int32),
            pltpu.VMEM((K,), jnp.float32),
            pltpu.VMEM((rpt,), jnp.float32),
            pltpu.VMEM_SHARED((Npad,), jnp.float32),
        ] + [pltpu.SemaphoreType.DMA for _ in range(nb)],
    )
    def deg_kernel(dst_hbm, ones_hbm, zeros_hbm, out_hbm, dst_vb, ones_v, row_v,
                   acc_sh, *sems):
        c = lax.axis_index("c")
        s = lax.axis_index("s")
        wid = s * NC + c
        row0 = s * rpt
        chunk0 = wid * per_w
        chunk_end = jnp.minimum(chunk0 + per_w, n_chunks)
        pltpu.sync_copy(zeros_hbm, row_v)
        pltpu.sync_copy(row_v, acc_sh.at[pl.ds(row0, rpt)])
        pltpu.sync_copy(ones_hbm, ones_v)
        plsc.subcore_barrier()

        pltpu.sync_copy(dst_hbm.at[pl.ds(chunk0, per_w)], dst_vb)

        def scat(b, q):
            return pltpu.make_async_copy(ones_v, acc_sh.at[dst_vb.at[q]],
                                         sems[b])

        for b in range(nb):
            @pl.when(chunk0 + b < chunk_end)
            def _(b=b):
                scat(b, b).start(add=True)

        def body(r, carry):
            for b in range(nb):
                q = r * nb + b
                qn = q + nb

                @pl.when(chunk0 + q < chunk_end)
                def _(b=b, q=q):
                    scat(b, q).wait()

                @pl.when((qn < per_w) & (chunk0 + qn < chunk_end))
                def _(b=b, qn=qn):
                    scat(b, qn).start(add=True)

            return carry

        lax.fori_loop(0, per_w // nb, body, None)
        plsc.subcore_barrier()
        pltpu.sync_copy(acc_sh.at[pl.ds(row0, rpt)], row_v)
        pltpu.sync_copy(row_v, out_hbm.at[pl.ds(c * Npad + row0, rpt)])

    return deg_kernel


def _make_agg_kernel(Npad, D, n_chunks, rpt, nb=2):
    per_w = _round_up(_cdiv(n_chunks, NW), max(8, nb))
    if per_w <= 64:
        slabs = [(0, per_w)]
        slab_max = per_w
    else:
        half = _round_up(per_w // 2, nb)
        slabs = [(0, half), (half, per_w - half)]
        slab_max = half
    mesh = plsc.VectorSubcoreMesh(core_axis_name="c", subcore_axis_name="s")

    scratch = (
        [pltpu.VMEM((slab_max, K), jnp.int32),
         pltpu.VMEM((slab_max, K), jnp.int32)]
        + [pltpu.VMEM((K, D), jnp.float32) for _ in range(nb)]
        + [pltpu.VMEM_SHARED((Npad, D), jnp.float32)]
        + [pltpu.SemaphoreType.DMA for _ in range(2 * nb)]
    )

    @functools.partial(
        pl.kernel,
        out_type=jax.ShapeDtypeStruct((NC, Npad, D), jnp.float32),
        mesh=mesh,
        scratch_types=scratch,
    )
    def agg_kernel(h_hbm, src_hbm, dst_hbm, zeros_hbm, out_hbm, src_vb, dst_vb,
                   *rest):
        rows = rest[:nb]
        acc_sh = rest[nb]
        sem_g = rest[nb + 1:nb + 1 + nb]
        sem_s = rest[nb + 1 + nb:]
        c = lax.axis_index("c")
        s = lax.axis_index("s")
        wid = s * NC + c
        row0 = s * rpt
        chunk0 = wid * per_w
        chunk_end = jnp.minimum(chunk0 + per_w, n_chunks)

        pltpu.sync_copy(zeros_hbm, acc_sh.at[pl.ds(row0, rpt)])
        plsc.subcore_barrier()

        def gather(b, ql):
            return pltpu.make_async_copy(h_hbm.at[src_vb.at[ql]], rows[b],
                                         sem_g[b])

        def scatter(b, ql):
            return pltpu.make_async_copy(rows[b], acc_sh.at[dst_vb.at[ql]],
                                         sem_s[b])

        for seg0, seg_len in slabs:
            pltpu.sync_copy(src_hbm.at[pl.ds(chunk0 + seg0, seg_len)],
                            src_vb.at[pl.ds(0, seg_len)])
            pltpu.sync_copy(dst_hbm.at[pl.ds(chunk0 + seg0, seg_len)],
                            dst_vb.at[pl.ds(0, seg_len)])

            for b in range(nb):
                @pl.when(chunk0 + seg0 + b < chunk_end)
                def _(b=b):
                    gather(b, b).start()

            def body(r, carry, seg0=seg0, seg_len=seg_len):
                for b in range(nb):
                    ql = r * nb + b
                    q = seg0 + ql

                    @pl.when(chunk0 + q < chunk_end)
                    def _(b=b, ql=ql):
                        gather(b, ql).wait()
                        scatter(b, ql).start(add=True)

                for b in range(nb):
                    ql = r * nb + b
                    q = seg0 + ql
                    qln = ql + nb

                    @pl.when(chunk0 + q < chunk_end)
                    def _(b=b, ql=ql):
                        scatter(b, ql).wait()

                    @pl.when((qln < seg_len) & (chunk0 + seg0 + qln < chunk_end))
                    def _(b=b, qln=qln):
                        gather(b, qln).start()

                return carry

            lax.fori_loop(0, seg_len // nb, body, None)

        plsc.subcore_barrier()
        pltpu.sync_copy(acc_sh.at[pl.ds(row0, rpt)], out_hbm.at[c, pl.ds(row0, rpt)])

    return agg_kernel


def _tc_matmul(xp, W1):
    Npad, DIN = xp.shape
    DH = W1.shape[1]

    def body(x_ref, w_ref, y_ref):
        y_ref[...] = jnp.dot(x_ref[...], w_ref[...],
                             preferred_element_type=jnp.float32)

    return pl.pallas_call(
        body,
        out_shape=jax.ShapeDtypeStruct((Npad, DH), jnp.float32),
    )(xp, W1)


def _tc_dinv_scale(degp0, degp1, y1):
    Npad, DH = y1.shape

    def body(d0_ref, d1_ref, y_ref, dinv_ref, ys_ref):
        deg = d0_ref[...] + d1_ref[...] + 1.0
        dinv = lax.rsqrt(deg)
        dinv_ref[...] = dinv
        ys_ref[...] = y_ref[...] * dinv

    return pl.pallas_call(
        body,
        out_shape=[
            jax.ShapeDtypeStruct((Npad, 1), jnp.float32),
            jax.ShapeDtypeStruct((Npad, DH), jnp.float32),
        ],
    )(degp0, degp1, y1)


def _tc_relu(p0, p1, y1, dinv, b1):
    Npad, DH = y1.shape

    def body(p0_ref, p1_ref, y_ref, dinv_ref, b_ref, out_ref):
        dinv = dinv_ref[...]
        t = p0_ref[...] + p1_ref[...] + y_ref[...] * dinv
        z = jnp.maximum(dinv * t + b_ref[...], 0.0)
        out_ref[...] = z * dinv

    return pl.pallas_call(
        body,
        out_shape=jax.ShapeDtypeStruct((Npad, DH), jnp.float32),
    )(p0, p1, y1, dinv, b1)


def _tc_logsoftmax(p0, p1, zs, dinv, W2p, b2, dout):
    Npad, DH = zs.shape
    Dp2 = W2p.shape[1]

    def body(p0_ref, p1_ref, z_ref, dinv_ref, w_ref, b_ref, out_ref):
        agg = dinv_ref[...] * (p0_ref[...] + p1_ref[...] + z_ref[...])
        t = jnp.dot(agg, w_ref[...], preferred_element_type=jnp.float32)
        t = t + b_ref[...]
        col = lax.broadcasted_iota(jnp.int32, t.shape, 1)
        valid = col < dout
        t = jnp.where(valid, t, jnp.float32(-1e30))
        m = jnp.max(t, axis=1, keepdims=True)
        e = jnp.where(valid, jnp.exp(t - m), 0.0)
        lse = jnp.log(jnp.sum(e, axis=1, keepdims=True))
        out_ref[...] = t - m - lse

    return pl.pallas_call(
        body,
        out_shape=jax.ShapeDtypeStruct((Npad, Dp2), jnp.float32),
    )(p0, p1, zs, dinv, W2p, b2)


def kernel(x, edge_index, W1, b1, W2, b2):
    N, DIN = x.shape
    DH = W1.shape[1]
    DOUT = W2.shape[1]
    E = edge_index.shape[1]

    Npad = _cdiv(N, 128) * 128
    rpt = Npad // NS
    n_chunks = _cdiv(E, K)
    Ep = n_chunks * K
    Dp2 = _cdiv(DOUT, 16) * 16
    nb = 2

    src = edge_index[0].astype(jnp.int32)
    dst = edge_index[1].astype(jnp.int32)
    if Ep != E:
        pad = N + (jnp.arange(Ep - E, dtype=jnp.int32) % (Npad - N))
        src = jnp.concatenate([src, pad])
        dst = jnp.concatenate([dst, pad])
    src2 = src.reshape(n_chunks, K)
    dst2 = dst.reshape(n_chunks, K)
    per_w = _round_up(_cdiv(n_chunks, NW), max(8, nb))
    n_chunks_pad = NW * per_w
    if n_chunks_pad != n_chunks:
        src2 = jnp.pad(src2, ((0, n_chunks_pad - n_chunks), (0, 0)))
        dst2 = jnp.pad(dst2, ((0, n_chunks_pad - n_chunks), (0, 0)))

    xp = jnp.pad(x, ((0, Npad - N), (0, 0)))
    W2p = jnp.pad(W2, ((0, 0), (0, Dp2 - DOUT)))
    b1r = b1.reshape(1, DH)
    b2r = jnp.pad(b2, (0, Dp2 - DOUT)).reshape(1, Dp2)
    ones_k = jnp.ones((K,), jnp.float32)
    zeros_deg = jnp.zeros((rpt,), jnp.float32)
    zeros_h = jnp.zeros((rpt, DH), jnp.float32)

    y1 = _tc_matmul(xp, W1)
    degp = _make_deg_kernel(Npad, n_chunks, rpt)(dst2, ones_k, zeros_deg)
    degp0 = degp[:Npad].reshape(Npad, 1)
    degp1 = degp[Npad:].reshape(Npad, 1)

    dinv, ys = _tc_dinv_scale(degp0, degp1, y1)

    agg_fn = _make_agg_kernel(Npad, DH, n_chunks, rpt, nb=nb)
    aggp = agg_fn(ys, src2, dst2, zeros_h)
    zs = _tc_relu(aggp[0], aggp[1], y1, dinv, b1r)

    agg2 = agg_fn(zs, src2, dst2, zeros_h)
    out = _tc_logsoftmax(agg2[0], agg2[1], zs, dinv, W2p, b2r, DOUT)

    return out[:N, :DOUT]


# speedup vs baseline: 1.1061x; 1.0058x over previous
import functools

import jax
import jax.numpy as jnp
from jax import lax
from jax.experimental import pallas as pl
from jax.experimental.pallas import tpu as pltpu
from jax.experimental.pallas import tpu_sc as plsc

NC = 2
NS = 16
NW = NC * NS
K = 128


def _cdiv(a, b):
    return (a + b - 1) // b


def _round_up(a, b):
    return _cdiv(a, b) * b


def _make_deg_kernel(Npad, n_chunks, rpt, nb=4):
    per_w = _round_up(_cdiv(n_chunks, NW), nb)
    mesh = plsc.VectorSubcoreMesh(core_axis_name="c", subcore_axis_name="s")

    @functools.partial(
        pl.kernel,
        out_type=jax.ShapeDtypeStruct((NC * Npad,), jnp.float32),
        mesh=mesh,
        scratch_types=[
            pltpu.VMEM((per_w, K), jnp.int32),
            pltpu.VMEM((K,), jnp.float32),
            pltpu.VMEM((rpt,), jnp.float32),
            pltpu.VMEM_SHARED((Npad,), jnp.float32),
        ] + [pltpu.SemaphoreType.DMA for _ in range(nb)],
    )
    def deg_kernel(dst_hbm, ones_hbm, zeros_hbm, out_hbm, dst_vb, ones_v, row_v,
                   acc_sh, *sems):
        c = lax.axis_index("c")
        s = lax.axis_index("s")
        wid = s * NC + c
        row0 = s * rpt
        chunk0 = wid * per_w
        chunk_end = jnp.minimum(chunk0 + per_w, n_chunks)
        pltpu.sync_copy(zeros_hbm, row_v)
        pltpu.sync_copy(row_v, acc_sh.at[pl.ds(row0, rpt)])
        pltpu.sync_copy(ones_hbm, ones_v)
        plsc.subcore_barrier()

        pltpu.sync_copy(dst_hbm.at[pl.ds(chunk0, per_w)], dst_vb)

        def scat(b, q):
            return pltpu.make_async_copy(ones_v, acc_sh.at[dst_vb.at[q]],
                                         sems[b])

        for b in range(nb):
            @pl.when(chunk0 + b < chunk_end)
            def _(b=b):
                scat(b, b).start(add=True)

        def body(r, carry):
            for b in range(nb):
                q = r * nb + b
                qn = q + nb

                @pl.when(chunk0 + q < chunk_end)
                def _(b=b, q=q):
                    scat(b, q).wait()

                @pl.when((qn < per_w) & (chunk0 + qn < chunk_end))
                def _(b=b, qn=qn):
                    scat(b, qn).start(add=True)

            return carry

        lax.fori_loop(0, per_w // nb, body, None)
        plsc.subcore_barrier()
        pltpu.sync_copy(acc_sh.at[pl.ds(row0, rpt)], row_v)
        pltpu.sync_copy(row_v, out_hbm.at[pl.ds(c * Npad + row0, rpt)])

    return deg_kernel


def _make_agg_kernel(Npad, D, n_chunks, rpt, nb=2):
    per_w = _round_up(_cdiv(n_chunks, NW), max(8, nb))
    if per_w <= 64:
        slabs = [(0, per_w)]
        slab_max = per_w
    else:
        half = _round_up(per_w // 2, nb)
        slabs = [(0, half), (half, per_w - half)]
        slab_max = half
    mesh = plsc.VectorSubcoreMesh(core_axis_name="c", subcore_axis_name="s")

    scratch = (
        [pltpu.VMEM((slab_max, K), jnp.int32),
         pltpu.VMEM((slab_max, K), jnp.int32)]
        + [pltpu.VMEM((K, D), jnp.float32) for _ in range(nb)]
        + [pltpu.VMEM_SHARED((Npad, D), jnp.float32)]
        + [pltpu.SemaphoreType.DMA for _ in range(2 * nb)]
    )

    @functools.partial(
        pl.kernel,
        out_type=jax.ShapeDtypeStruct((NC, Npad, D), jnp.float32),
        mesh=mesh,
        scratch_types=scratch,
    )
    def agg_kernel(h_hbm, src_hbm, dst_hbm, zeros_hbm, out_hbm, src_vb, dst_vb,
                   *rest):
        rows = rest[:nb]
        acc_sh = rest[nb]
        sem_g = rest[nb + 1:nb + 1 + nb]
        sem_s = rest[nb + 1 + nb:]
        c = lax.axis_index("c")
        s = lax.axis_index("s")
        wid = s * NC + c
        row0 = s * rpt
        chunk0 = wid * per_w
        chunk_end = jnp.minimum(chunk0 + per_w, n_chunks)

        pltpu.sync_copy(zeros_hbm, acc_sh.at[pl.ds(row0, rpt)])
        plsc.subcore_barrier()

        def gather(b, ql):
            return pltpu.make_async_copy(h_hbm.at[src_vb.at[ql]], rows[b],
                                         sem_g[b])

        def scatter(b, ql):
            return pltpu.make_async_copy(rows[b], acc_sh.at[dst_vb.at[ql]],
                                         sem_s[b])

        for seg0, seg_len in slabs:
            pltpu.sync_copy(src_hbm.at[pl.ds(chunk0 + seg0, seg_len)],
                            src_vb.at[pl.ds(0, seg_len)])
            pltpu.sync_copy(dst_hbm.at[pl.ds(chunk0 + seg0, seg_len)],
                            dst_vb.at[pl.ds(0, seg_len)])

            for b in range(nb):
                @pl.when(chunk0 + seg0 + b < chunk_end)
                def _(b=b):
                    gather(b, b).start()

            def body(r, carry, seg0=seg0, seg_len=seg_len):
                for b in range(nb):
                    ql = r * nb + b
                    q = seg0 + ql

                    @pl.when(chunk0 + q < chunk_end)
                    def _(b=b, ql=ql):
                        gather(b, ql).wait()
                        scatter(b, ql).start(add=True)

                for b in range(nb):
                    ql = r * nb + b
                    q = seg0 + ql
                    qln = ql + nb

                    @pl.when(chunk0 + q < chunk_end)
                    def _(b=b, ql=ql):
                        scatter(b, ql).wait()

                    @pl.when((qln < seg_len) & (chunk0 + seg0 + qln < chunk_end))
                    def _(b=b, qln=qln):
                        gather(b, qln).start()

                return carry

            lax.fori_loop(0, seg_len // nb, body, None)

        plsc.subcore_barrier()
        pltpu.sync_copy(acc_sh.at[pl.ds(row0, rpt)], out_hbm.at[c, pl.ds(row0, rpt)])

    return agg_kernel


def _tc_scale_matmul(degp0, degp1, xp, W1):
    Npad, DIN = xp.shape
    DH = W1.shape[1]

    def body(d0_ref, d1_ref, x_ref, w_ref, dinv_ref, ys_ref):
        deg = d0_ref[...] + d1_ref[...] + 1.0
        dinv = lax.rsqrt(deg)
        dinv_ref[...] = dinv
        ys_ref[...] = jnp.dot(x_ref[...] * dinv, w_ref[...],
                              preferred_element_type=jnp.float32)

    return pl.pallas_call(
        body,
        out_shape=[
            jax.ShapeDtypeStruct((Npad, 1), jnp.float32),
            jax.ShapeDtypeStruct((Npad, DH), jnp.float32),
        ],
    )(degp0, degp1, xp, W1)


def _tc_relu(p0, p1, ys, dinv, b1):
    Npad, DH = ys.shape

    def body(p0_ref, p1_ref, y_ref, dinv_ref, b_ref, out_ref):
        dinv = dinv_ref[...]
        t = p0_ref[...] + p1_ref[...] + y_ref[...]
        z = jnp.maximum(dinv * t + b_ref[...], 0.0)
        out_ref[...] = z * dinv

    return pl.pallas_call(
        body,
        out_shape=jax.ShapeDtypeStruct((Npad, DH), jnp.float32),
    )(p0, p1, ys, dinv, b1)


def _tc_logsoftmax(p0, p1, zs, dinv, W2p, b2, dout):
    Npad, DH = zs.shape
    Dp2 = W2p.shape[1]

    def body(p0_ref, p1_ref, z_ref, dinv_ref, w_ref, b_ref, out_ref):
        agg = dinv_ref[...] * (p0_ref[...] + p1_ref[...] + z_ref[...])
        t = jnp.dot(agg, w_ref[...], preferred_element_type=jnp.float32)
        t = t + b_ref[...]
        col = lax.broadcasted_iota(jnp.int32, t.shape, 1)
        valid = col < dout
        t = jnp.where(valid, t, jnp.float32(-1e30))
        m = jnp.max(t, axis=1, keepdims=True)
        e = jnp.where(valid, jnp.exp(t - m), 0.0)
        lse = jnp.log(jnp.sum(e, axis=1, keepdims=True))
        out_ref[...] = t - m - lse

    return pl.pallas_call(
        body,
        out_shape=jax.ShapeDtypeStruct((Npad, Dp2), jnp.float32),
    )(p0, p1, zs, dinv, W2p, b2)


def kernel(x, edge_index, W1, b1, W2, b2):
    N, DIN = x.shape
    DH = W1.shape[1]
    DOUT = W2.shape[1]
    E = edge_index.shape[1]

    Npad = _cdiv(N, 128) * 128
    rpt = Npad // NS
    n_chunks = _cdiv(E, K)
    Ep = n_chunks * K
    Dp2 = _cdiv(DOUT, 16) * 16
    nb = 2

    src = edge_index[0].astype(jnp.int32)
    dst = edge_index[1].astype(jnp.int32)
    if Ep != E:
        pad = N + (jnp.arange(Ep - E, dtype=jnp.int32) % (Npad - N))
        src = jnp.concatenate([src, pad])
        dst = jnp.concatenate([dst, pad])
    src2 = src.reshape(n_chunks, K)
    dst2 = dst.reshape(n_chunks, K)
    per_w = _round_up(_cdiv(n_chunks, NW), max(8, nb))
    n_chunks_pad = NW * per_w
    if n_chunks_pad != n_chunks:
        src2 = jnp.pad(src2, ((0, n_chunks_pad - n_chunks), (0, 0)))
        dst2 = jnp.pad(dst2, ((0, n_chunks_pad - n_chunks), (0, 0)))

    xp = jnp.pad(x, ((0, Npad - N), (0, 0)))
    W2p = jnp.pad(W2, ((0, 0), (0, Dp2 - DOUT)))
    b1r = b1.reshape(1, DH)
    b2r = jnp.pad(b2, (0, Dp2 - DOUT)).reshape(1, Dp2)
    ones_k = jnp.ones((K,), jnp.float32)
    zeros_deg = jnp.zeros((rpt,), jnp.float32)
    zeros_h = jnp.zeros((rpt, DH), jnp.float32)

    degp = _make_deg_kernel(Npad, n_chunks, rpt)(dst2, ones_k, zeros_deg)
    degp0 = degp[:Npad].reshape(Npad, 1)
    degp1 = degp[Npad:].reshape(Npad, 1)

    dinv, ys = _tc_scale_matmul(degp0, degp1, xp, W1)

    agg_fn = _make_agg_kernel(Npad, DH, n_chunks, rpt, nb=nb)
    aggp = agg_fn(ys, src2, dst2, zeros_h)
    zs = _tc_relu(aggp[0], aggp[1], ys, dinv, b1r)

    agg2 = agg_fn(zs, src2, dst2, zeros_h)
    out = _tc_logsoftmax(agg2[0], agg2[1], zs, dinv, W2p, b2r, DOUT)

    return out[:N, :DOUT]


# speedup vs baseline: 1.1142x; 1.0074x over previous
import functools

import jax
import jax.numpy as jnp
from jax import lax
from jax.experimental import pallas as pl
from jax.experimental.pallas import tpu as pltpu
from jax.experimental.pallas import tpu_sc as plsc

NC = 2
NS = 16
NW = NC * NS
K = 128


def _cdiv(a, b):
    return (a + b - 1) // b


def _round_up(a, b):
    return _cdiv(a, b) * b


def _make_deg_kernel(Npad, n_chunks, rpt, nb=4):
    per_w = _round_up(_cdiv(n_chunks, NW), nb)
    mesh = plsc.VectorSubcoreMesh(core_axis_name="c", subcore_axis_name="s")

    @functools.partial(
        pl.kernel,
        out_type=jax.ShapeDtypeStruct((NC * Npad,), jnp.float32),
        mesh=mesh,
        scratch_types=[
            pltpu.VMEM((per_w, K), jnp.int32),
            pltpu.VMEM((K,), jnp.float32),
            pltpu.VMEM((rpt,), jnp.float32),
            pltpu.VMEM_SHARED((Npad,), jnp.float32),
        ] + [pltpu.SemaphoreType.DMA for _ in range(nb)],
    )
    def deg_kernel(dst_hbm, ones_hbm, zeros_hbm, out_hbm, dst_vb, ones_v, row_v,
                   acc_sh, *sems):
        c = lax.axis_index("c")
        s = lax.axis_index("s")
        wid = s * NC + c
        row0 = s * rpt
        chunk0 = wid * per_w
        chunk_end = jnp.minimum(chunk0 + per_w, n_chunks)
        pltpu.sync_copy(zeros_hbm, row_v)
        pltpu.sync_copy(row_v, acc_sh.at[pl.ds(row0, rpt)])
        pltpu.sync_copy(ones_hbm, ones_v)
        plsc.subcore_barrier()

        pltpu.sync_copy(dst_hbm.at[pl.ds(chunk0, per_w)], dst_vb)

        def scat(b, q):
            return pltpu.make_async_copy(ones_v, acc_sh.at[dst_vb.at[q]],
                                         sems[b])

        for b in range(nb):
            @pl.when(chunk0 + b < chunk_end)
            def _(b=b):
                scat(b, b).start(add=True)

        def body(r, carry):
            for b in range(nb):
                q = r * nb + b
                qn = q + nb

                @pl.when(chunk0 + q < chunk_end)
                def _(b=b, q=q):
                    scat(b, q).wait()

                @pl.when((qn < per_w) & (chunk0 + qn < chunk_end))
                def _(b=b, qn=qn):
                    scat(b, qn).start(add=True)

            return carry

        lax.fori_loop(0, per_w // nb, body, None)
        plsc.subcore_barrier()
        pltpu.sync_copy(acc_sh.at[pl.ds(row0, rpt)], row_v)
        pltpu.sync_copy(row_v, out_hbm.at[pl.ds(c * Npad + row0, rpt)])

    return deg_kernel


def _make_agg_kernel(Npad, D, n_chunks, rpt, nb=2):
    per_w = _round_up(_cdiv(n_chunks, NW), max(8, nb))
    if per_w <= 64:
        slabs = [(0, per_w)]
        slab_max = per_w
    else:
        half = _round_up(per_w // 2, nb)
        slabs = [(0, half), (half, per_w - half)]
        slab_max = half
    mesh = plsc.VectorSubcoreMesh(core_axis_name="c", subcore_axis_name="s")

    scratch = (
        [pltpu.VMEM((slab_max, K), jnp.int32),
         pltpu.VMEM((slab_max, K), jnp.int32)]
        + [pltpu.VMEM((K, D), jnp.float32) for _ in range(nb)]
        + [pltpu.VMEM_SHARED((Npad, D), jnp.float32)]
        + [pltpu.SemaphoreType.DMA for _ in range(2 * nb + 1)]
    )

    @functools.partial(
        pl.kernel,
        out_type=jax.ShapeDtypeStruct((NC, Npad, D), jnp.float32),
        mesh=mesh,
        scratch_types=scratch,
    )
    def agg_kernel(h_hbm, src_hbm, dst_hbm, zeros_hbm, out_hbm, src_vb, dst_vb,
                   *rest):
        rows = rest[:nb]
        acc_sh = rest[nb]
        sem_g = rest[nb + 1:nb + 1 + nb]
        sem_s = rest[nb + 1 + nb:nb + 1 + 2 * nb]
        sem_z = rest[nb + 1 + 2 * nb]
        c = lax.axis_index("c")
        s = lax.axis_index("s")
        wid = s * NC + c
        row0 = s * rpt
        chunk0 = wid * per_w
        chunk_end = jnp.minimum(chunk0 + per_w, n_chunks)

        zc = pltpu.make_async_copy(zeros_hbm, acc_sh.at[pl.ds(row0, rpt)],
                                   sem_z)
        zc.start()
        pltpu.sync_copy(src_hbm.at[pl.ds(chunk0, slabs[0][1])],
                        src_vb.at[pl.ds(0, slabs[0][1])])
        pltpu.sync_copy(dst_hbm.at[pl.ds(chunk0, slabs[0][1])],
                        dst_vb.at[pl.ds(0, slabs[0][1])])
        zc.wait()
        plsc.subcore_barrier()

        def gather(b, ql):
            return pltpu.make_async_copy(h_hbm.at[src_vb.at[ql]], rows[b],
                                         sem_g[b])

        def scatter(b, ql):
            return pltpu.make_async_copy(rows[b], acc_sh.at[dst_vb.at[ql]],
                                         sem_s[b])

        for si, (seg0, seg_len) in enumerate(slabs):
            if si > 0:
                pltpu.sync_copy(src_hbm.at[pl.ds(chunk0 + seg0, seg_len)],
                                src_vb.at[pl.ds(0, seg_len)])
                pltpu.sync_copy(dst_hbm.at[pl.ds(chunk0 + seg0, seg_len)],
                                dst_vb.at[pl.ds(0, seg_len)])

            for b in range(nb):
                @pl.when(chunk0 + seg0 + b < chunk_end)
                def _(b=b):
                    gather(b, b).start()

            def body(r, carry, seg0=seg0, seg_len=seg_len):
                for b in range(nb):
                    ql = r * nb + b
                    q = seg0 + ql

                    @pl.when(chunk0 + q < chunk_end)
                    def _(b=b, ql=ql):
                        gather(b, ql).wait()
                        scatter(b, ql).start(add=True)

                for b in range(nb):
                    ql = r * nb + b
                    q = seg0 + ql
                    qln = ql + nb

                    @pl.when(chunk0 + q < chunk_end)
                    def _(b=b, ql=ql):
                        scatter(b, ql).wait()

                    @pl.when((qln < seg_len) & (chunk0 + seg0 + qln < chunk_end))
                    def _(b=b, qln=qln):
                        gather(b, qln).start()

                return carry

            lax.fori_loop(0, seg_len // nb, body, None)

        plsc.subcore_barrier()
        pltpu.sync_copy(acc_sh.at[pl.ds(row0, rpt)], out_hbm.at[c, pl.ds(row0, rpt)])

    return agg_kernel


def _tc_scale_matmul(degp0, degp1, xp, W1):
    Npad, DIN = xp.shape
    DH = W1.shape[1]

    def body(d0_ref, d1_ref, x_ref, w_ref, dinv_ref, ys_ref):
        deg = d0_ref[...] + d1_ref[...] + 1.0
        dinv = lax.rsqrt(deg)
        dinv_ref[...] = dinv
        ys_ref[...] = jnp.dot(x_ref[...] * dinv, w_ref[...],
                              preferred_element_type=jnp.float32)

    return pl.pallas_call(
        body,
        out_shape=[
            jax.ShapeDtypeStruct((Npad, 1), jnp.float32),
            jax.ShapeDtypeStruct((Npad, DH), jnp.float32),
        ],
    )(degp0, degp1, xp, W1)


def _tc_relu(p0, p1, ys, dinv, b1):
    Npad, DH = ys.shape

    def body(p0_ref, p1_ref, y_ref, dinv_ref, b_ref, out_ref):
        dinv = dinv_ref[...]
        t = p0_ref[...] + p1_ref[...] + y_ref[...]
        z = jnp.maximum(dinv * t + b_ref[...], 0.0)
        out_ref[...] = z * dinv

    return pl.pallas_call(
        body,
        out_shape=jax.ShapeDtypeStruct((Npad, DH), jnp.float32),
    )(p0, p1, ys, dinv, b1)


def _tc_logsoftmax(p0, p1, zs, dinv, W2p, b2, dout):
    Npad, DH = zs.shape
    Dp2 = W2p.shape[1]

    def body(p0_ref, p1_ref, z_ref, dinv_ref, w_ref, b_ref, out_ref):
        agg = dinv_ref[...] * (p0_ref[...] + p1_ref[...] + z_ref[...])
        t = jnp.dot(agg, w_ref[...], preferred_element_type=jnp.float32)
        t = t + b_ref[...]
        col = lax.broadcasted_iota(jnp.int32, t.shape, 1)
        valid = col < dout
        t = jnp.where(valid, t, jnp.float32(-1e30))
        m = jnp.max(t, axis=1, keepdims=True)
        e = jnp.where(valid, jnp.exp(t - m), 0.0)
        lse = jnp.log(jnp.sum(e, axis=1, keepdims=True))
        out_ref[...] = t - m - lse

    return pl.pallas_call(
        body,
        out_shape=jax.ShapeDtypeStruct((Npad, Dp2), jnp.float32),
    )(p0, p1, zs, dinv, W2p, b2)


def kernel(x, edge_index, W1, b1, W2, b2):
    N, DIN = x.shape
    DH = W1.shape[1]
    DOUT = W2.shape[1]
    E = edge_index.shape[1]

    Npad = _cdiv(N, 128) * 128
    rpt = Npad // NS
    n_chunks = _cdiv(E, K)
    Ep = n_chunks * K
    Dp2 = _cdiv(DOUT, 16) * 16
    nb = 2

    src = edge_index[0].astype(jnp.int32)
    dst = edge_index[1].astype(jnp.int32)
    if Ep != E:
        pad = N + (jnp.arange(Ep - E, dtype=jnp.int32) % (Npad - N))
        src = jnp.concatenate([src, pad])
        dst = jnp.concatenate([dst, pad])
    src2 = src.reshape(n_chunks, K)
    dst2 = dst.reshape(n_chunks, K)
    per_w = _round_up(_cdiv(n_chunks, NW), max(8, nb))
    n_chunks_pad = NW * per_w
    if n_chunks_pad != n_chunks:
        src2 = jnp.pad(src2, ((0, n_chunks_pad - n_chunks), (0, 0)))
        dst2 = jnp.pad(dst2, ((0, n_chunks_pad - n_chunks), (0, 0)))

    xp = jnp.pad(x, ((0, Npad - N), (0, 0)))
    W2p = jnp.pad(W2, ((0, 0), (0, Dp2 - DOUT)))
    b1r = b1.reshape(1, DH)
    b2r = jnp.pad(b2, (0, Dp2 - DOUT)).reshape(1, Dp2)
    ones_k = jnp.ones((K,), jnp.float32)
    zeros_deg = jnp.zeros((rpt,), jnp.float32)
    zeros_h = jnp.zeros((rpt, DH), jnp.float32)

    degp = _make_deg_kernel(Npad, n_chunks, rpt)(dst2, ones_k, zeros_deg)
    degp0 = degp[:Npad].reshape(Npad, 1)
    degp1 = degp[Npad:].reshape(Npad, 1)

    dinv, ys = _tc_scale_matmul(degp0, degp1, xp, W1)

    agg_fn = _make_agg_kernel(Npad, DH, n_chunks, rpt, nb=nb)
    aggp = agg_fn(ys, src2, dst2, zeros_h)
    zs = _tc_relu(aggp[0], aggp[1], ys, dinv, b1r)

    agg2 = agg_fn(zs, src2, dst2, zeros_h)
    out = _tc_logsoftmax(agg2[0], agg2[1], zs, dinv, W2p, b2r, DOUT)

    return out[:N, :DOUT]


# speedup vs baseline: 1.3692x; 1.2288x over previous
import functools

import jax
import jax.numpy as jnp
from jax import lax
from jax.experimental import pallas as pl
from jax.experimental.pallas import tpu as pltpu
from jax.experimental.pallas import tpu_sc as plsc

NC = 2
NS = 16
NW = NC * NS
K = 128


def _cdiv(a, b):
    return (a + b - 1) // b


def _round_up(a, b):
    return _cdiv(a, b) * b


def _make_deg_kernel(Npad, n_chunks, rpt, nb=4):
    per_w = _round_up(_cdiv(n_chunks, NW), nb)
    mesh = plsc.VectorSubcoreMesh(core_axis_name="c", subcore_axis_name="s")

    @functools.partial(
        pl.kernel,
        out_type=jax.ShapeDtypeStruct((NC * Npad,), jnp.float32),
        mesh=mesh,
        scratch_types=[
            pltpu.VMEM((per_w, K), jnp.int32),
            pltpu.VMEM((K,), jnp.float32),
            pltpu.VMEM((rpt,), jnp.float32),
            pltpu.VMEM_SHARED((Npad,), jnp.float32),
        ] + [pltpu.SemaphoreType.DMA for _ in range(nb)],
    )
    def deg_kernel(dst_hbm, ones_hbm, zeros_hbm, out_hbm, dst_vb, ones_v, row_v,
                   acc_sh, *sems):
        c = lax.axis_index("c")
        s = lax.axis_index("s")
        wid = s * NC + c
        row0 = s * rpt
        chunk0 = wid * per_w
        chunk_end = jnp.minimum(chunk0 + per_w, n_chunks)
        pltpu.sync_copy(zeros_hbm, row_v)
        pltpu.sync_copy(row_v, acc_sh.at[pl.ds(row0, rpt)])
        pltpu.sync_copy(ones_hbm, ones_v)
        plsc.subcore_barrier()

        pltpu.sync_copy(dst_hbm.at[pl.ds(chunk0, per_w)], dst_vb)

        def scat(b, q):
            return pltpu.make_async_copy(ones_v, acc_sh.at[dst_vb.at[q]],
                                         sems[b])

        for b in range(nb):
            @pl.when(chunk0 + b < chunk_end)
            def _(b=b):
                scat(b, b).start(add=True)

        def body(r, carry):
            for b in range(nb):
                q = r * nb + b
                qn = q + nb

                @pl.when(chunk0 + q < chunk_end)
                def _(b=b, q=q):
                    scat(b, q).wait()

                @pl.when((qn < per_w) & (chunk0 + qn < chunk_end))
                def _(b=b, qn=qn):
                    scat(b, qn).start(add=True)

            return carry

        lax.fori_loop(0, per_w // nb, body, None)
        plsc.subcore_barrier()
        pltpu.sync_copy(acc_sh.at[pl.ds(row0, rpt)], row_v)
        pltpu.sync_copy(row_v, out_hbm.at[pl.ds(c * Npad + row0, rpt)])

    return deg_kernel


def _make_agg_kernel(Npad, D, n_chunks, rpt, nb=3):
    per_w = _round_up(_cdiv(n_chunks, NW), nb)
    mesh = plsc.VectorSubcoreMesh(core_axis_name="c", subcore_axis_name="s")

    scratch = (
        [pltpu.VMEM((2 * K,), jnp.int32) for _ in range(nb)]
        + [pltpu.VMEM((K, D), jnp.float32) for _ in range(nb)]
        + [pltpu.VMEM_SHARED((Npad, D), jnp.float32)]
        + [pltpu.SemaphoreType.DMA for _ in range(3 * nb + 1)]
    )

    @functools.partial(
        pl.kernel,
        out_type=jax.ShapeDtypeStruct((NC, Npad, D), jnp.float32),
        mesh=mesh,
        scratch_types=scratch,
    )
    def agg_kernel(h_hbm, sd_hbm, zeros_hbm, out_hbm, *rest):
        idxb = rest[:nb]
        rows = rest[nb:2 * nb]
        acc_sh = rest[2 * nb]
        sem_i = rest[2 * nb + 1:2 * nb + 1 + nb]
        sem_g = rest[2 * nb + 1 + nb:2 * nb + 1 + 2 * nb]
        sem_s = rest[2 * nb + 1 + 2 * nb:2 * nb + 1 + 3 * nb]
        sem_z = rest[2 * nb + 1 + 3 * nb]
        c = lax.axis_index("c")
        s = lax.axis_index("s")
        wid = s * NC + c
        row0 = s * rpt
        chunk0 = wid * per_w
        chunk_end = jnp.minimum(chunk0 + per_w, n_chunks)

        def idx_fetch(b, q):
            return pltpu.make_async_copy(sd_hbm.at[chunk0 + q], idxb[b],
                                         sem_i[b])

        def gather(b):
            return pltpu.make_async_copy(
                h_hbm.at[idxb[b].at[pl.ds(0, K)]], rows[b], sem_g[b])

        def scatter(b):
            return pltpu.make_async_copy(
                rows[b], acc_sh.at[idxb[b].at[pl.ds(K, K)]], sem_s[b])

        zc = pltpu.make_async_copy(zeros_hbm, acc_sh.at[pl.ds(row0, rpt)],
                                   sem_z)
        zc.start()

        @pl.when(chunk0 < chunk_end)
        def _():
            idx_fetch(0, 0).start()

        zc.wait()
        plsc.subcore_barrier()

        def body(r, carry):
            for b in range(nb):
                q = r * nb + b
                bp1 = (b + 1) % nb
                bp2 = (b + 2) % nb

                @pl.when(chunk0 + q < chunk_end)
                def _(b=b, q=q):
                    idx_fetch(b, q).wait()
                    gather(b).start()

                @pl.when((q >= 1) & (chunk0 + q - 1 < chunk_end))
                def _(bp2=bp2, q=q):
                    gather(bp2).wait()
                    scatter(bp2).start(add=True)

                @pl.when((q >= 2) & (chunk0 + q - 2 < chunk_end))
                def _(bp1=bp1, q=q):
                    scatter(bp1).wait()

                @pl.when((q + 1 < per_w) & (chunk0 + q + 1 < chunk_end))
                def _(bp1=bp1, q=q):
                    idx_fetch(bp1, q + 1).start()

            return carry

        lax.fori_loop(0, per_w // nb, body, None)

        qL = per_w - 1
        bL = qL % nb

        @pl.when(chunk0 + qL < chunk_end)
        def _():
            gather(bL).wait()
            scatter(bL).start(add=True)

        @pl.when((qL >= 1) & (chunk0 + qL - 1 < chunk_end))
        def _():
            scatter((qL - 1) % nb).wait()

        @pl.when(chunk0 + qL < chunk_end)
        def _():
            scatter(bL).wait()

        plsc.subcore_barrier()
        pltpu.sync_copy(acc_sh.at[pl.ds(row0, rpt)], out_hbm.at[c, pl.ds(row0, rpt)])

    return agg_kernel


def _tc_scale_matmul(degp0, degp1, xp, W1):
    Npad, DIN = xp.shape
    DH = W1.shape[1]

    def body(d0_ref, d1_ref, x_ref, w_ref, dinv_ref, ys_ref):
        deg = d0_ref[...] + d1_ref[...] + 1.0
        dinv = lax.rsqrt(deg)
        dinv_ref[...] = dinv
        ys_ref[...] = jnp.dot(x_ref[...] * dinv, w_ref[...],
                              preferred_element_type=jnp.float32)

    return pl.pallas_call(
        body,
        out_shape=[
            jax.ShapeDtypeStruct((Npad, 1), jnp.float32),
            jax.ShapeDtypeStruct((Npad, DH), jnp.float32),
        ],
    )(degp0, degp1, xp, W1)


def _tc_relu(p0, p1, ys, dinv, b1):
    Npad, DH = ys.shape

    def body(p0_ref, p1_ref, y_ref, dinv_ref, b_ref, out_ref):
        dinv = dinv_ref[...]
        t = p0_ref[...] + p1_ref[...] + y_ref[...]
        z = jnp.maximum(dinv * t + b_ref[...], 0.0)
        out_ref[...] = z * dinv

    return pl.pallas_call(
        body,
        out_shape=jax.ShapeDtypeStruct((Npad, DH), jnp.float32),
    )(p0, p1, ys, dinv, b1)


def _tc_logsoftmax(p0, p1, zs, dinv, W2p, b2, dout):
    Npad, DH = zs.shape
    Dp2 = W2p.shape[1]

    def body(p0_ref, p1_ref, z_ref, dinv_ref, w_ref, b_ref, out_ref):
        agg = dinv_ref[...] * (p0_ref[...] + p1_ref[...] + z_ref[...])
        t = jnp.dot(agg, w_ref[...], preferred_element_type=jnp.float32)
        t = t + b_ref[...]
        col = lax.broadcasted_iota(jnp.int32, t.shape, 1)
        valid = col < dout
        t = jnp.where(valid, t, jnp.float32(-1e30))
        m = jnp.max(t, axis=1, keepdims=True)
        e = jnp.where(valid, jnp.exp(t - m), 0.0)
        lse = jnp.log(jnp.sum(e, axis=1, keepdims=True))
        out_ref[...] = t - m - lse

    return pl.pallas_call(
        body,
        out_shape=jax.ShapeDtypeStruct((Npad, Dp2), jnp.float32),
    )(p0, p1, zs, dinv, W2p, b2)


def kernel(x, edge_index, W1, b1, W2, b2):
    N, DIN = x.shape
    DH = W1.shape[1]
    DOUT = W2.shape[1]
    E = edge_index.shape[1]

    Npad = _cdiv(N, 128) * 128
    rpt = Npad // NS
    n_chunks = _cdiv(E, K)
    Ep = n_chunks * K
    Dp2 = _cdiv(DOUT, 16) * 16
    nb = 3

    src = edge_index[0].astype(jnp.int32)
    dst = edge_index[1].astype(jnp.int32)
    if Ep != E:
        pad = N + (jnp.arange(Ep - E, dtype=jnp.int32) % (Npad - N))
        src = jnp.concatenate([src, pad])
        dst = jnp.concatenate([dst, pad])
    src2 = src.reshape(n_chunks, K)
    dst2 = dst.reshape(n_chunks, K)
    per_w = max(_round_up(_cdiv(n_chunks, NW), 4),
                _round_up(_cdiv(n_chunks, NW), nb))
    n_chunks_pad = NW * per_w
    if n_chunks_pad != n_chunks:
        src2 = jnp.pad(src2, ((0, n_chunks_pad - n_chunks), (0, 0)))
        dst2 = jnp.pad(dst2, ((0, n_chunks_pad - n_chunks), (0, 0)))
    sd2 = jnp.concatenate([src2, dst2], axis=1)

    xp = jnp.pad(x, ((0, Npad - N), (0, 0)))
    W2p = jnp.pad(W2, ((0, 0), (0, Dp2 - DOUT)))
    b1r = b1.reshape(1, DH)
    b2r = jnp.pad(b2, (0, Dp2 - DOUT)).reshape(1, Dp2)
    ones_k = jnp.ones((K,), jnp.float32)
    zeros_deg = jnp.zeros((rpt,), jnp.float32)
    zeros_h = jnp.zeros((rpt, DH), jnp.float32)

    degp = _make_deg_kernel(Npad, n_chunks, rpt)(dst2, ones_k, zeros_deg)
    degp0 = degp[:Npad].reshape(Npad, 1)
    degp1 = degp[Npad:].reshape(Npad, 1)

    dinv, ys = _tc_scale_matmul(degp0, degp1, xp, W1)

    agg_fn = _make_agg_kernel(Npad, DH, n_chunks, rpt, nb=nb)
    aggp = agg_fn(ys, sd2, zeros_h)
    zs = _tc_relu(aggp[0], aggp[1], ys, dinv, b1r)

    agg2 = agg_fn(zs, sd2, zeros_h)
    out = _tc_logsoftmax(agg2[0], agg2[1], zs, dinv, W2p, b2r, DOUT)

    return out[:N, :DOUT]
